# Initial kernel scaffold; baseline (speedup 1.0000x reference)
import functools

import jax
import jax.numpy as jnp
from jax import lax
from jax.experimental import pallas as pl
from jax.experimental.pallas import tpu as pltpu
from jax.experimental.pallas import tpu_sc as plsc

N = 10000
F1 = 128
C = 40
F2 = 48

_NC, _NS = 2, 16
_NW = _NC * _NS
_K = 80
_RPT = N // _NS
_ZR = 125
_DF = 16
_BN = 2000


def _make_prop(F, nch):
    mesh = plsc.VectorSubcoreMesh(core_axis_name="c", subcore_axis_name="s")

    @functools.partial(
        pl.kernel,
        out_type=jax.ShapeDtypeStruct((_NC, N, F), jnp.float32),
        mesh=mesh,
        scratch_types=[
            pltpu.VMEM((nch, _K), jnp.int32),
            pltpu.VMEM((nch, _K), jnp.int32),
            pltpu.VMEM((_K, F), jnp.float32),
            pltpu.VMEM((_ZR, F), jnp.float32),
            pltpu.VMEM_SHARED((N, F), jnp.float32),
            pltpu.SemaphoreType.DMA,
        ],
    )
    def prop(hs, srcg, dstg, zrow, out, src_v, dst_v, rows_v, stage_v, acc, sem):
        c = lax.axis_index("c")
        s = lax.axis_index("s")
        w = c * _NS + s
        pltpu.sync_copy(zrow, stage_v)
        for j in range(_RPT // _ZR):
            pltpu.sync_copy(stage_v, acc.at[pl.ds(s * _RPT + j * _ZR, _ZR)])
        pltpu.sync_copy(srcg.at[w], src_v)
        pltpu.sync_copy(dstg.at[w], dst_v)
        plsc.subcore_barrier()

        def body(j, carry):
            pltpu.async_copy(hs.at[src_v.at[j]], rows_v, sem).wait()
            pltpu.sync_copy(rows_v, acc.at[dst_v.at[j]], add=True)
            return carry

        lax.fori_loop(0, nch, body, 0)
        plsc.subcore_barrier()
        for j in range(_RPT // _ZR):
            r0 = s * _RPT + j * _ZR
            pltpu.sync_copy(acc.at[pl.ds(r0, _ZR)], stage_v)
            pltpu.sync_copy(stage_v, out.at[c, pl.ds(r0, _ZR)])

    return prop


def _make_degree(nch):
    mesh = plsc.VectorSubcoreMesh(core_axis_name="c", subcore_axis_name="s")

    @functools.partial(
        pl.kernel,
        out_type=jax.ShapeDtypeStruct((_NC, N, _DF), jnp.float32),
        mesh=mesh,
        scratch_types=[
            pltpu.VMEM((nch, _K), jnp.int32),
            pltpu.VMEM((_K, _DF), jnp.float32),
            pltpu.VMEM((_ZR, _DF), jnp.float32),
            pltpu.VMEM_SHARED((N, _DF), jnp.float32),
            pltpu.SemaphoreType.DMA,
        ],
    )
    def deg(dstg, ones, zrow, out, dst_v, ones_v, stage_v, acc, sem):
        c = lax.axis_index("c")
        s = lax.axis_index("s")
        w = c * _NS + s
        pltpu.sync_copy(ones, ones_v)
        pltpu.sync_copy(zrow, stage_v)
        for j in range(_RPT // _ZR):
            pltpu.sync_copy(stage_v, acc.at[pl.ds(s * _RPT + j * _ZR, _ZR)])
        pltpu.sync_copy(dstg.at[w], dst_v)
        plsc.subcore_barrier()

        def body(j, carry):
            pltpu.sync_copy(ones_v, acc.at[dst_v.at[j]], add=True)
            return carry

        lax.fori_loop(0, nch, body, 0)
        plsc.subcore_barrier()
        for j in range(_RPT // _ZR):
            r0 = s * _RPT + j * _ZR
            pltpu.sync_copy(acc.at[pl.ds(r0, _ZR)], stage_v)
            pltpu.sync_copy(stage_v, out.at[c, pl.ds(r0, _ZR)])

    return deg


def _tc1(degp, x, w1):
    def body(degp_ref, x_ref, w1_ref, h1s_ref, dinv_ref):
        deg = degp_ref[0] + degp_ref[1] + 1.0
        dinv = lax.rsqrt(deg)
        h = jnp.dot(x_ref[...], w1_ref[...], preferred_element_type=jnp.float32)
        h1s_ref[...] = h * dinv[:, 0:1]
        dinv_ref[...] = dinv

    return pl.pallas_call(
        body,
        grid=(N // _BN,),
        in_specs=[
            pl.BlockSpec((_NC, _BN, _DF), lambda i: (0, i, 0)),
            pl.BlockSpec((_BN, F1), lambda i: (i, 0)),
            pl.BlockSpec((F1, F1), lambda i: (0, 0)),
        ],
        out_specs=[
            pl.BlockSpec((_BN, F1), lambda i: (i, 0)),
            pl.BlockSpec((_BN, _DF), lambda i: (i, 0)),
        ],
        out_shape=[
            jax.ShapeDtypeStruct((N, F1), jnp.float32),
            jax.ShapeDtypeStruct((N, _DF), jnp.float32),
        ],
    )(degp, x, w1)


def _tc2(p1, h1s, dinv, b1, w2p):
    def body(p1_ref, h1s_ref, dinv_ref, b1_ref, w2p_ref, h2s_ref):
        di = dinv_ref[...][:, 0:1]
        acc = p1_ref[0] + p1_ref[1] + h1s_ref[...]
        a = jnp.maximum(acc * di + b1_ref[...], 0.0)
        h2s_ref[...] = jnp.dot(
            a, w2p_ref[...], preferred_element_type=jnp.float32) * di

    return pl.pallas_call(
        body,
        grid=(N // _BN,),
        in_specs=[
            pl.BlockSpec((_NC, _BN, F1), lambda i: (0, i, 0)),
            pl.BlockSpec((_BN, F1), lambda i: (i, 0)),
            pl.BlockSpec((_BN, _DF), lambda i: (i, 0)),
            pl.BlockSpec((1, F1), lambda i: (0, 0)),
            pl.BlockSpec((F1, F2), lambda i: (0, 0)),
        ],
        out_specs=pl.BlockSpec((_BN, F2), lambda i: (i, 0)),
        out_shape=jax.ShapeDtypeStruct((N, F2), jnp.float32),
    )(p1, h1s, dinv, b1.reshape(1, F1), w2p)


def _tc3(p2, h2s, dinv, b2p):
    def body(p2_ref, h2s_ref, dinv_ref, b2p_ref, out_ref):
        di = dinv_ref[...][:, 0:1]
        z = (p2_ref[0] + p2_ref[1] + h2s_ref[...]) * di + b2p_ref[...]
        zc = z[:, :C]
        m = jnp.max(zc, axis=1, keepdims=True)
        lse = jnp.log(jnp.sum(jnp.exp(zc - m), axis=1, keepdims=True)) + m
        out_ref[...] = zc - lse

    return pl.pallas_call(
        body,
        grid=(N // _BN,),
        in_specs=[
            pl.BlockSpec((_NC, _BN, F2), lambda i: (0, i, 0)),
            pl.BlockSpec((_BN, F2), lambda i: (i, 0)),
            pl.BlockSpec((_BN, _DF), lambda i: (i, 0)),
            pl.BlockSpec((1, F2), lambda i: (0, 0)),
        ],
        out_specs=pl.BlockSpec((_BN, C), lambda i: (i, 0)),
        out_shape=jax.ShapeDtypeStruct((N, C), jnp.float32),
    )(p2, h2s, dinv, b2p.reshape(1, F2))


def kernel(x, edge_index, W1, b1, W2, b2):
    e = edge_index.shape[1]
    nch = e // (_NW * _K)
    src = edge_index[0].reshape(_NW, nch, _K)
    dst = edge_index[1].reshape(_NW, nch, _K)
    zrow1 = jnp.zeros((_ZR, F1), jnp.float32)
    zrow2 = jnp.zeros((_ZR, F2), jnp.float32)
    zrowd = jnp.zeros((_ZR, _DF), jnp.float32)
    onesd = jnp.ones((_K, _DF), jnp.float32)
    w2p = jnp.pad(W2, ((0, 0), (0, F2 - C)))
    b2p = jnp.pad(b2, (0, F2 - C))

    degp = _make_degree(nch)(dst, onesd, zrowd)
    h1s, dinv = _tc1(degp, x, W1)
    p1 = _make_prop(F1, nch)(h1s, src, dst, zrow1)
    h2s = _tc2(p1, h1s, dinv, b1, w2p)
    p2 = _make_prop(F2, nch)(h2s, src, dst, zrow2)
    return _tc3(p2, h2s, dinv, b2p)


# speedup vs baseline: 4.7313x; 4.7313x over previous
import functools

import jax
import jax.numpy as jnp
from jax import lax
from jax.experimental import pallas as pl
from jax.experimental.pallas import tpu as pltpu
from jax.experimental.pallas import tpu_sc as plsc

N = 10000
NP = 10240
F1 = 128
C = 40
F2 = 48

_NC, _NS = 2, 16
_NW = _NC * _NS
_K = 80
_GR = 128
_DW = 16
_DR = NP // 16
_BN = 2048


def _make_prop(nch, passes, rows=NP):
    rng = -(-rows // (passes * 128)) * 128
    lens = [min(rng, rows - i * rng) for i in range(passes)]
    ar = rng + _GR
    zpt = ar // _NS
    mesh = plsc.VectorSubcoreMesh(core_axis_name="c", subcore_axis_name="s")

    @functools.partial(
        pl.kernel,
        out_type=jax.ShapeDtypeStruct((_NC, rows, F1), jnp.float32),
        mesh=mesh,
        scratch_types=[
            pltpu.VMEM((nch, _K), jnp.int32),
            pltpu.VMEM((nch, _K), jnp.int32),
            pltpu.VMEM((_K,), jnp.int32),
            pltpu.VMEM((_K, F1), jnp.float32),
            pltpu.VMEM((zpt, F1), jnp.float32),
            pltpu.VMEM_SHARED((ar, F1), jnp.float32),
            pltpu.SemaphoreType.DMA,
        ],
    )
    def prop(ts, srcg, dstg, zrow, out,
             src_v, dst_v, loc_v, rows_v, stage_v, acc, sem):
        c = lax.axis_index("c")
        s = lax.axis_index("s")
        w = c * _NS + s
        pltpu.sync_copy(srcg.at[w], src_v)
        pltpu.sync_copy(dstg.at[w], dst_v)

        for p in range(passes):
            base = p * rng
            plen = lens[p]
            wpt = plen // _NS
            pltpu.sync_copy(zrow, stage_v)
            pltpu.sync_copy(stage_v, acc.at[pl.ds(s * zpt, zpt)])
            plsc.subcore_barrier()

            def body(j, carry):
                pltpu.async_copy(ts.at[src_v.at[j]], rows_v, sem).wait()
                for t in range(_K // 16):
                    dv = dst_v[j, pl.ds(t * 16, 16)]
                    loc = dv - base
                    ok = (loc >= 0) & (loc < plen)
                    loc_v[pl.ds(t * 16, 16)] = jnp.where(ok, loc, rng + t * 8)
                pltpu.sync_copy(rows_v, acc.at[loc_v], add=True)
                return carry

            lax.fori_loop(0, nch, body, 0)
            plsc.subcore_barrier()
            r0 = s * wpt
            pltpu.sync_copy(acc.at[pl.ds(r0, wpt)], stage_v.at[pl.ds(0, wpt)])
            pltpu.sync_copy(stage_v.at[pl.ds(0, wpt)],
                            out.at[c, pl.ds(base + r0, wpt)])
            plsc.subcore_barrier()

    return prop


def _tc1(d0, d1, x):
    def body(d0_ref, d1_ref, x_ref, xs_ref, dinv_ref):
        deg = d0_ref[...][:, 0:1] + d1_ref[...][:, 0:1] + 1.0
        dinv = lax.rsqrt(deg)
        xs_ref[...] = x_ref[...] * dinv
        dinv_ref[...] = jnp.broadcast_to(dinv, (_BN, _DW))

    return pl.pallas_call(
        body,
        grid=(NP // _BN,),
        in_specs=[
            pl.BlockSpec((_BN, 8), lambda i: (i, 0)),
            pl.BlockSpec((_BN, 8), lambda i: (i, 0)),
            pl.BlockSpec((_BN, F1), lambda i: (i, 0)),
        ],
        out_specs=[
            pl.BlockSpec((_BN, F1), lambda i: (i, 0)),
            pl.BlockSpec((_BN, _DW), lambda i: (i, 0)),
        ],
        out_shape=[
            jax.ShapeDtypeStruct((NP, F1), jnp.float32),
            jax.ShapeDtypeStruct((NP, _DW), jnp.float32),
        ],
    )(d0, d1, x)


def _tc2(p1, xs, dinv, b1, w1):
    def body(p1_ref, xs_ref, dinv_ref, b1_ref, w1_ref, as_ref):
        di = dinv_ref[...][:, 0:1]
        px = (p1_ref[0] + p1_ref[1] + xs_ref[...]) * di
        h = jnp.dot(px, w1_ref[...], preferred_element_type=jnp.float32)
        as_ref[...] = jnp.maximum(h + b1_ref[...], 0.0) * di

    return pl.pallas_call(
        body,
        grid=(NP // _BN,),
        in_specs=[
            pl.BlockSpec((_NC, _BN, F1), lambda i: (0, i, 0)),
            pl.BlockSpec((_BN, F1), lambda i: (i, 0)),
            pl.BlockSpec((_BN, _DW), lambda i: (i, 0)),
            pl.BlockSpec((1, F1), lambda i: (0, 0)),
            pl.BlockSpec((F1, F1), lambda i: (0, 0)),
        ],
        out_specs=pl.BlockSpec((_BN, F1), lambda i: (i, 0)),
        out_shape=jax.ShapeDtypeStruct((NP, F1), jnp.float32),
    )(p1, xs, dinv, b1.reshape(1, F1), w1)


def _tc3(p2, as_, dinv, b2p, w2p):
    def body(p2_ref, as_ref, dinv_ref, b2p_ref, w2p_ref, out_ref):
        di = dinv_ref[...][:, 0:1]
        pa = (p2_ref[0] + p2_ref[1] + as_ref[...]) * di
        z = jnp.dot(pa, w2p_ref[...],
                    preferred_element_type=jnp.float32) + b2p_ref[...]
        zc = z[:, :C]
        m = jnp.max(zc, axis=1, keepdims=True)
        lse = jnp.log(jnp.sum(jnp.exp(zc - m), axis=1, keepdims=True)) + m
        out_ref[...] = zc - lse

    return pl.pallas_call(
        body,
        grid=(NP // _BN,),
        in_specs=[
            pl.BlockSpec((_NC, _BN, F1), lambda i: (0, i, 0)),
            pl.BlockSpec((_BN, F1), lambda i: (i, 0)),
            pl.BlockSpec((_BN, _DW), lambda i: (i, 0)),
            pl.BlockSpec((1, F2), lambda i: (0, 0)),
            pl.BlockSpec((F1, F2), lambda i: (0, 0)),
        ],
        out_specs=pl.BlockSpec((_BN, C), lambda i: (i, 0)),
        out_shape=jax.ShapeDtypeStruct((NP, C), jnp.float32),
    )(p2, as_, dinv, b2p.reshape(1, F2), w2p)


def kernel(x, edge_index, W1, b1, W2, b2):
    e = edge_index.shape[1]
    nch = e // (_NW * _K)
    src32 = edge_index[0].reshape(_NW, nch, _K)
    dst32 = edge_index[1].reshape(_NW, nch, _K)
    zrow = jnp.zeros(((3456 + _GR) // _NS, F1), jnp.float32)
    zrowd = jnp.zeros(((_DR + _GR) // _NS, F1), jnp.float32)
    pats = jnp.repeat(jnp.eye(16, dtype=jnp.float32), 8, axis=1)
    dmod = (edge_index[1] & 15).reshape(_NW, nch, _K)
    ddiv = (edge_index[1] >> 4).reshape(_NW, nch, _K)
    w2p = jnp.pad(W2, ((0, 0), (0, F2 - C)))
    b2p = jnp.pad(b2, (0, F2 - C))
    xp = jnp.pad(x, ((0, NP - N), (0, 0)))

    degp = _make_prop(nch, 1, _DR)(pats, dmod, ddiv, zrowd)
    xs, dinv = _tc1(degp[0].reshape(NP, 8), degp[1].reshape(NP, 8), xp)
    prop = _make_prop(nch, 3)
    p1 = prop(xs, src32, dst32, zrow)
    as_ = _tc2(p1, xs, dinv, b1, W1)
    p2 = prop(as_, src32, dst32, zrow)
    return _tc3(p2, as_, dinv, b2p, w2p)[:N]
